# baseline (speedup 1.0000x reference)
import functools

import jax
import jax.numpy as jnp
from jax import lax
from jax.experimental import pallas as pl
from jax.experimental.pallas import tpu as pltpu
from jax.experimental.pallas import tpu_sc as plsc

_B, _H, _D = 16384, 200, 32
_NC, _NS = 2, 16
_NW = _NC * _NS
_RPW = _B // _NW
_K = 40
_NP = _H // _K
_OB = 32
_RUN = 8
_NBUF = 8


def _emb_pool_body(x_hbm, emb_hbm, out_hbm, idx_v, acc_v, out_v, sem_g):
    wid = lax.axis_index("s") * _NC + lax.axis_index("c")
    row0 = wid * _RPW

    pltpu.sync_copy(x_hbm.at[pl.ds(row0 * _H, _RPW * _H)], idx_v)

    zb = jnp.zeros((32,), jnp.bfloat16)
    zf = jnp.zeros((16,), jnp.float32)
    def zero_body(j, carry):
        for p in range(_NBUF):
            acc_v[p, j, :] = zb
        return carry

    lax.fori_loop(0, _K, zero_body, 0)

    def fire(r, b):
        for k in range(_NP):
            pltpu.async_copy(
                emb_hbm.at[idx_v.at[pl.ds(r * _H + k * _K, _K)]],
                acc_v.at[b], sem_g, add=True)

    def drain(r, b):
        for k in range(_NP):
            pltpu.make_async_copy(
                emb_hbm.at[idx_v.at[pl.ds(r * _H + k * _K, _K)]],
                acc_v.at[b], sem_g).wait()

    for rr in range(_NBUF - 1):
        fire(rr, rr)

    ii2 = lax.iota(jnp.int32, 16) * 2

    def row_body(r, carry):
        p = lax.rem(r, _NBUF)

        @pl.when(r + _NBUF - 1 < _RPW)
        def _():
            fire(r + _NBUF - 1, lax.rem(r + _NBUF - 1, _NBUF))

        drain(r, p)

        def red_body(j, acc):
            a0, a1, a2, a3 = acc
            for k in range(_RUN):
                jj = j * _RUN + k
                ev, od = plsc.unpack(
                    acc_v[p, jj, :], format=plsc.PackFormat.INTERLEAVED)
                if k % 2 == 0:
                    a0 = a0 + ev
                    a2 = a2 + od
                else:
                    a1 = a1 + ev
                    a3 = a3 + od
                acc_v[p, jj, :] = zb
            return a0, a1, a2, a3

        a0, a1, a2, a3 = lax.fori_loop(
            0, _K // _RUN, red_body, (zf, zf, zf, zf))
        rl = lax.rem(r, _OB)
        base = rl * _D + ii2
        plsc.store_scatter(out_v, [base], a0 + a1)
        plsc.store_scatter(out_v, [base + 1], a2 + a3)

        @pl.when(rl == _OB - 1)
        def _():
            pltpu.sync_copy(
                out_v, out_hbm.at[pl.ds((row0 + r - (_OB - 1)) * _D, _OB * _D)])

        return carry

    lax.fori_loop(0, _RPW, row_body, 0)


_emb_pool = functools.partial(
    pl.kernel,
    out_type=jax.ShapeDtypeStruct((_B * _D,), jnp.float32),
    mesh=plsc.VectorSubcoreMesh(core_axis_name="c", subcore_axis_name="s"),
    compiler_params=pltpu.CompilerParams(
        use_tc_tiling_on_sc=False, needs_layout_passes=False),
    scratch_types=[
        pltpu.VMEM((_RPW * _H,), jnp.int32),
        pltpu.VMEM((_NBUF, _K, _D), jnp.bfloat16),
        pltpu.VMEM((_OB * _D,), jnp.float32),
        pltpu.SemaphoreType.DMA,
    ],
)(_emb_pool_body)


@jax.jit
def kernel(x, action_emb):
    out_flat = _emb_pool(x.reshape(-1), action_emb.astype(jnp.bfloat16))
    return out_flat.reshape(_B, _D)


# speedup vs baseline: 1.1921x; 1.1921x over previous
import functools

import jax
import jax.numpy as jnp
from jax import lax
from jax.experimental import pallas as pl
from jax.experimental.pallas import tpu as pltpu
from jax.experimental.pallas import tpu_sc as plsc

_B, _H, _D = 16384, 200, 32
_NC, _NS = 2, 16
_NW = _NC * _NS
_RPW = _B // _NW
_K = 40
_NP = _H // _K
_OB = 32
_RUN = 8
_NBUF = 8
_V = 100000
_VS = _V // _NS
_IBR = 32
_NBLK = _RPW // _IBR
_IBW = _IBR * _H


def _emb_pool_body(x_hbm, emb_hbm, out_hbm, idx_v, acc_v, out_v, emb_sh,
                   sem_g, sem_i):
    wid = lax.axis_index("s") * _NC + lax.axis_index("c")
    sid = lax.axis_index("s")
    row0 = wid * _RPW

    pltpu.sync_copy(emb_hbm.at[pl.ds(sid * _VS, _VS), :],
                    emb_sh.at[pl.ds(sid * _VS, _VS), :])

    zb = jnp.zeros((32,), jnp.bfloat16)
    zf = jnp.zeros((16,), jnp.float32)

    def zero_body(j, carry):
        for p in range(_NBUF):
            acc_v[p, j, :] = zb
        return carry

    lax.fori_loop(0, _K, zero_body, 0)
    plsc.subcore_barrier()

    def stage_idx(blk, buf):
        return pltpu.async_copy(
            x_hbm.at[pl.ds((row0 + blk * _IBR) * _H, _IBW)],
            idx_v.at[buf], sem_i)

    def fire_at(r):
        blk = r // _IBR
        rl = lax.rem(r, _IBR)
        for k in range(_NP):
            pltpu.async_copy(
                emb_sh.at[idx_v.at[lax.rem(blk, 3),
                                   pl.ds(rl * _H + k * _K, _K)]],
                acc_v.at[lax.rem(r, _NBUF)], sem_g, add=True)

    def drain_at(r):
        blk = r // _IBR
        rl = lax.rem(r, _IBR)
        for k in range(_NP):
            pltpu.make_async_copy(
                emb_sh.at[idx_v.at[lax.rem(blk, 3),
                                   pl.ds(rl * _H + k * _K, _K)]],
                acc_v.at[lax.rem(r, _NBUF)], sem_g).wait()

    stage_idx(0, 0).wait()
    stage_idx(1, 1)

    for rr in range(_NBUF - 1):
        fire_at(rr)

    ii2 = lax.iota(jnp.int32, 16) * 2

    def blk_body(blk, carry):
        @pl.when(blk + 1 < _NBLK)
        def _():
            pltpu.make_async_copy(
                x_hbm.at[pl.ds((row0 + (blk + 1) * _IBR) * _H, _IBW)],
                idx_v.at[lax.rem(blk + 1, 3)], sem_i).wait()

        @pl.when(blk + 2 < _NBLK)
        def _():
            stage_idx(blk + 2, lax.rem(blk + 2, 3))

        def row_body(rl, carry):
            r = blk * _IBR + rl

            @pl.when(r + _NBUF - 1 < _RPW)
            def _():
                fire_at(r + _NBUF - 1)

            drain_at(r)
            p = lax.rem(r, _NBUF)

            def red_body(j, acc):
                a0, a1, a2, a3 = acc
                for k in range(_RUN):
                    jj = j * _RUN + k
                    ev, od = plsc.unpack(
                        acc_v[p, jj, :], format=plsc.PackFormat.INTERLEAVED)
                    if k % 2 == 0:
                        a0 = a0 + ev
                        a2 = a2 + od
                    else:
                        a1 = a1 + ev
                        a3 = a3 + od
                    acc_v[p, jj, :] = zb
                return a0, a1, a2, a3

            a0, a1, a2, a3 = lax.fori_loop(
                0, _K // _RUN, red_body, (zf, zf, zf, zf))
            base = rl * _D + ii2
            plsc.store_scatter(out_v, [base], a0 + a1)
            plsc.store_scatter(out_v, [base + 1], a2 + a3)
            return carry

        lax.fori_loop(0, _IBR, row_body, 0)
        pltpu.sync_copy(
            out_v, out_hbm.at[pl.ds((row0 + blk * _IBR) * _D, _OB * _D)])
        return carry

    lax.fori_loop(0, _NBLK, blk_body, 0)


_emb_pool = functools.partial(
    pl.kernel,
    out_type=jax.ShapeDtypeStruct((_B * _D,), jnp.float32),
    mesh=plsc.VectorSubcoreMesh(core_axis_name="c", subcore_axis_name="s"),
    compiler_params=pltpu.CompilerParams(
        use_tc_tiling_on_sc=False, needs_layout_passes=False),
    scratch_types=[
        pltpu.VMEM((3, _IBW), jnp.int32),
        pltpu.VMEM((_NBUF, _K, _D), jnp.bfloat16),
        pltpu.VMEM((_OB * _D,), jnp.float32),
        pltpu.VMEM_SHARED((_V, _D), jnp.bfloat16),
        pltpu.SemaphoreType.DMA,
        pltpu.SemaphoreType.DMA,
    ],
)(_emb_pool_body)


@jax.jit
def kernel(x, action_emb):
    out_flat = _emb_pool(x.reshape(-1), action_emb.astype(jnp.bfloat16))
    return out_flat.reshape(_B, _D)
